# baseline (speedup 1.0000x reference)
import functools

import jax
import jax.numpy as jnp
from jax import lax
from jax.experimental import pallas as pl
from jax.experimental.pallas import tpu as pltpu, tpu_sc as plsc

_NC = 2
_NS = 16
_L = 16

_K = 1000
_Q = 1024
_D = 16

_K_SC = 600
_K_TC = _K - _K_SC

_NQB = 8
_QB = _Q // _NQB
_NG = _QB // _L
_NJ = 4
_KJ = _K_SC // _NJ
_KB = 25
_NCHUNK = _KJ // _KB

_KBT = 50
_TSTEPS = _K_TC // _KBT


def _sc_body(agg_hbm, t_hbm, outd_hbm, outv_hbm, buf, tvm, stage, merged,
             outd, outv, shared, sems):
    c = lax.axis_index("c")
    s = lax.axis_index("s")
    b = c * (_NQB // _NC) + s // _NJ
    j = s % _NJ
    q0 = b * _QB
    k_base = j * _KJ

    pltpu.sync_copy(t_hbm.at[:, pl.ds(q0, _QB)], tvm)

    def src(ci):
        return agg_hbm.at[pl.ds(k_base + ci * _KB, _KB), :, pl.ds(q0, _QB)]

    inf = jnp.full((_L,), jnp.inf, jnp.float32)
    zero = jnp.zeros((_L,), jnp.float32)
    carry = (inf, zero) * _NG

    pltpu.async_copy(src(0), buf.at[0], sems.at[0])

    def chunk_body(ci, carry):
        par = lax.rem(ci, 2)
        cur = buf.at[par]
        pltpu.make_async_copy(src(ci), cur, sems.at[par]).wait()

        @pl.when(ci + 1 < _NCHUNK)
        def _prefetch():
            pltpu.async_copy(src(ci + 1), buf.at[1 - par], sems.at[1 - par])

        out = list(carry)
        for g in range(_NG):
            tg = [tvm[d, pl.ds(g * _L, _L)] for d in range(_D)]

            def k_body(k, st, g=g, tg=tg, cur=cur):
                best, bval = st
                v0 = None
                acc_a = None
                acc_b = None
                for d in range(_D):
                    v = cur[k, d, pl.ds(g * _L, _L)]
                    if d == 0:
                        v0 = v
                    term = jnp.abs(v - tg[d])
                    if d % 2 == 0:
                        acc_a = term if acc_a is None else acc_a + term
                    else:
                        acc_b = term if acc_b is None else acc_b + term
                dist = acc_a + acc_b
                better = dist < best
                return (jnp.where(better, dist, best),
                        jnp.where(better, v0, bval))

            out[2 * g], out[2 * g + 1] = lax.fori_loop(
                0, _KB, k_body, (out[2 * g], out[2 * g + 1]), unroll=5)
        return tuple(out)

    carry = lax.fori_loop(0, _NCHUNK, chunk_body, carry)

    for g in range(_NG):
        stage[0, 0, pl.ds(g * _L, _L)] = carry[2 * g]
        stage[0, 0, pl.ds(_QB + g * _L, _L)] = carry[2 * g + 1]
    pltpu.sync_copy(stage, shared.at[pl.ds(s, 1)])
    plsc.subcore_barrier()

    @pl.when(j == 0)
    def _merge():
        pltpu.sync_copy(shared.at[pl.ds(s, _NJ)], merged)
        for g in range(_NG):
            bd = merged[0, 0, pl.ds(g * _L, _L)]
            bv = merged[0, 0, pl.ds(_QB + g * _L, _L)]
            for jj in range(1, _NJ):
                dd = merged[jj, 0, pl.ds(g * _L, _L)]
                vv = merged[jj, 0, pl.ds(_QB + g * _L, _L)]
                m = dd < bd
                bd = jnp.where(m, dd, bd)
                bv = jnp.where(m, vv, bv)
            outd[0, 0, pl.ds(g * _L, _L)] = bd
            outv[0, 0, pl.ds(g * _L, _L)] = bv
        pltpu.sync_copy(outd, outd_hbm.at[pl.ds(b, 1)])
        pltpu.sync_copy(outv, outv_hbm.at[pl.ds(b, 1)])


def _sc_call(agg_t, tensor_t):
    mesh = plsc.VectorSubcoreMesh(
        core_axis_name="c", subcore_axis_name="s",
        num_cores=_NC, num_subcores=_NS)
    return pl.kernel(
        _sc_body,
        out_type=(jax.ShapeDtypeStruct((_NQB, 1, _QB), jnp.float32),
                  jax.ShapeDtypeStruct((_NQB, 1, _QB), jnp.float32)),
        mesh=mesh,
        scratch_types=[
            pltpu.VMEM((2, _KB, _D, _QB), jnp.float32),
            pltpu.VMEM((_D, _QB), jnp.float32),
            pltpu.VMEM((1, 1, 2 * _QB), jnp.float32),
            pltpu.VMEM((_NJ, 1, 2 * _QB), jnp.float32),
            pltpu.VMEM((1, 1, _QB), jnp.float32),
            pltpu.VMEM((1, 1, _QB), jnp.float32),
            pltpu.VMEM_SHARED((_NS, 1, 2 * _QB), jnp.float32),
            pltpu.SemaphoreType.DMA((2,)),
        ],
        compiler_params=pltpu.CompilerParams(use_tc_tiling_on_sc=False,
                                             needs_layout_passes=False),
    )(agg_t, tensor_t)


def _pairwise_min(dc, vc):
    n = dc.shape[0]
    while n > 1:
        h = (n + 1) // 2
        a_d, a_v = dc[: n - h], vc[: n - h]
        b_d, b_v = dc[h:n], vc[h:n]
        w = b_d < a_d
        md = jnp.where(w, b_d, a_d)
        mv = jnp.where(w, b_v, a_v)
        if h > n - h:
            md = jnp.concatenate([md, dc[n - h:h]], axis=0)
            mv = jnp.concatenate([mv, vc[n - h:h]], axis=0)
        dc, vc = md, mv
        n = h
    return dc, vc


def _tc_body(t_ref, x_ref, d_ref, v_ref, bd, bv):
    i = pl.program_id(0)
    x = x_ref[...]
    t = t_ref[...]
    dist = jnp.sum(jnp.abs(x - t[None, :, :]), axis=1)
    val = x[:, 0, :]
    dc, vc = _pairwise_min(dist, val)

    @pl.when(i == 0)
    def _init():
        bd[...] = jnp.full((1, _Q), jnp.inf, jnp.float32)
        bv[...] = jnp.zeros((1, _Q), jnp.float32)

    pd = bd[...]
    pv = bv[...]
    w = dc < pd
    bd[...] = jnp.where(w, dc, pd)
    bv[...] = jnp.where(w, vc, pv)

    @pl.when(i == _TSTEPS - 1)
    def _emit():
        d_ref[...] = bd[...]
        v_ref[...] = bv[...]


def _tc_call(agg_t, tensor_t):
    return pl.pallas_call(
        _tc_body,
        grid=(_TSTEPS,),
        in_specs=[
            pl.BlockSpec((_D, _Q), lambda i: (0, 0)),
            pl.BlockSpec((_KBT, _D, _Q), lambda i: (_K_SC // _KBT + i, 0, 0)),
        ],
        out_specs=[
            pl.BlockSpec((1, _Q), lambda i: (0, 0)),
            pl.BlockSpec((1, _Q), lambda i: (0, 0)),
        ],
        out_shape=(jax.ShapeDtypeStruct((1, _Q), jnp.float32),
                   jax.ShapeDtypeStruct((1, _Q), jnp.float32)),
        scratch_shapes=[
            pltpu.VMEM((1, _Q), jnp.float32),
            pltpu.VMEM((1, _Q), jnp.float32),
        ],
    )(tensor_t, agg_t)


def _merge_body(sd_ref, sv_ref, td_ref, tv_ref, o_ref):
    sd = sd_ref[...]
    sv = sv_ref[...]
    w = td_ref[...] < sd
    o_ref[...] = jnp.where(w, tv_ref[...], sv)


def _merge_call(sc_d, sc_v, tc_d, tc_v):
    return pl.pallas_call(
        _merge_body,
        out_shape=jax.ShapeDtypeStruct((1, _Q), jnp.float32),
    )(sc_d, sc_v, tc_d, tc_v)


@jax.jit
def _run(agg_t, tensor_t):
    sc_d, sc_v = _sc_call(agg_t, tensor_t)
    tc_d, tc_v = _tc_call(agg_t, tensor_t)
    merged = _merge_call(sc_d.reshape(1, _Q), sc_v.reshape(1, _Q), tc_d, tc_v)
    return merged


def kernel(tensor, aggregated_values):
    k, q, d = aggregated_values.shape
    assert (k, q, d) == (_K, _Q, _D)
    agg_t = jnp.transpose(aggregated_values, (0, 2, 1))
    tensor_t = tensor.T
    out = _run(agg_t, tensor_t)
    return out.reshape(1, _Q, 1)


# speedup vs baseline: 2.1135x; 2.1135x over previous
import functools

import jax
import jax.numpy as jnp
from jax import lax
from jax.experimental import pallas as pl
from jax.experimental.pallas import tpu as pltpu, tpu_sc as plsc

_NC = 2
_NS = 16
_L = 16

_K = 1000
_Q = 1024
_D = 16

_K_SC = 500
_K_TC = _K - _K_SC

_NQB = 8
_QB = _Q // _NQB
_NG = _QB // _L
_NJ = 4
_KJ = _K_SC // _NJ
_KB = 25
_NCHUNK = _KJ // _KB

_KBT = 50
_TSTEPS = _K_TC // _KBT


def _sc_body(agg_hbm, t_hbm, outd_hbm, outv_hbm, buf, tvm, stage, merged,
             outd, outv, shared, sems):
    c = lax.axis_index("c")
    s = lax.axis_index("s")
    b = c * (_NQB // _NC) + s // _NJ
    j = s % _NJ
    q0 = b * _QB
    k_base = j * _KJ

    pltpu.sync_copy(t_hbm.at[:, pl.ds(q0, _QB)], tvm)

    def src(ci):
        return agg_hbm.at[pl.ds(k_base + ci * _KB, _KB), :, pl.ds(q0, _QB)]

    inf = jnp.full((_L,), jnp.inf, jnp.float32)
    zero = jnp.zeros((_L,), jnp.float32)
    carry = (inf, zero) * _NG

    pltpu.async_copy(src(0), buf.at[0], sems.at[0])

    def chunk_body(ci, carry):
        par = lax.rem(ci, 2)
        cur = buf.at[par]
        pltpu.make_async_copy(src(ci), cur, sems.at[par]).wait()

        @pl.when(ci + 1 < _NCHUNK)
        def _prefetch():
            pltpu.async_copy(src(ci + 1), buf.at[1 - par], sems.at[1 - par])

        out = list(carry)
        for g in range(_NG):
            tg = [tvm[d, pl.ds(g * _L, _L)] for d in range(_D)]

            def k_body(k, st, g=g, tg=tg, cur=cur):
                best, bval = st
                v0 = None
                acc_a = None
                acc_b = None
                for d in range(_D):
                    v = cur[k, d, pl.ds(g * _L, _L)]
                    if d == 0:
                        v0 = v
                    term = jnp.abs(v - tg[d])
                    if d % 2 == 0:
                        acc_a = term if acc_a is None else acc_a + term
                    else:
                        acc_b = term if acc_b is None else acc_b + term
                dist = acc_a + acc_b
                better = dist < best
                return (jnp.where(better, dist, best),
                        jnp.where(better, v0, bval))

            out[2 * g], out[2 * g + 1] = lax.fori_loop(
                0, _KB, k_body, (out[2 * g], out[2 * g + 1]), unroll=5)
        return tuple(out)

    carry = lax.fori_loop(0, _NCHUNK, chunk_body, carry)

    for g in range(_NG):
        stage[0, 0, pl.ds(g * _L, _L)] = carry[2 * g]
        stage[0, 0, pl.ds(_QB + g * _L, _L)] = carry[2 * g + 1]
    pltpu.sync_copy(stage, shared.at[pl.ds(s, 1)])
    plsc.subcore_barrier()

    @pl.when(j == 0)
    def _merge():
        pltpu.sync_copy(shared.at[pl.ds(s, _NJ)], merged)
        for g in range(_NG):
            bd = merged[0, 0, pl.ds(g * _L, _L)]
            bv = merged[0, 0, pl.ds(_QB + g * _L, _L)]
            for jj in range(1, _NJ):
                dd = merged[jj, 0, pl.ds(g * _L, _L)]
                vv = merged[jj, 0, pl.ds(_QB + g * _L, _L)]
                m = dd < bd
                bd = jnp.where(m, dd, bd)
                bv = jnp.where(m, vv, bv)
            outd[0, 0, pl.ds(g * _L, _L)] = bd
            outv[0, 0, pl.ds(g * _L, _L)] = bv
        pltpu.sync_copy(outd, outd_hbm.at[pl.ds(b, 1)])
        pltpu.sync_copy(outv, outv_hbm.at[pl.ds(b, 1)])


def _sc_call(agg_t, tensor_t):
    mesh = plsc.VectorSubcoreMesh(
        core_axis_name="c", subcore_axis_name="s",
        num_cores=_NC, num_subcores=_NS)
    return pl.kernel(
        _sc_body,
        out_type=(jax.ShapeDtypeStruct((_NQB, 1, _QB), jnp.float32),
                  jax.ShapeDtypeStruct((_NQB, 1, _QB), jnp.float32)),
        mesh=mesh,
        scratch_types=[
            pltpu.VMEM((2, _KB, _D, _QB), jnp.float32),
            pltpu.VMEM((_D, _QB), jnp.float32),
            pltpu.VMEM((1, 1, 2 * _QB), jnp.float32),
            pltpu.VMEM((_NJ, 1, 2 * _QB), jnp.float32),
            pltpu.VMEM((1, 1, _QB), jnp.float32),
            pltpu.VMEM((1, 1, _QB), jnp.float32),
            pltpu.VMEM_SHARED((_NS, 1, 2 * _QB), jnp.float32),
            pltpu.SemaphoreType.DMA((2,)),
        ],
    )(agg_t, tensor_t)


def _pairwise_min(dc, vc):
    n = dc.shape[0]
    while n > 1:
        h = (n + 1) // 2
        a_d, a_v = dc[: n - h], vc[: n - h]
        b_d, b_v = dc[h:n], vc[h:n]
        w = b_d < a_d
        md = jnp.where(w, b_d, a_d)
        mv = jnp.where(w, b_v, a_v)
        if h > n - h:
            md = jnp.concatenate([md, dc[n - h:h]], axis=0)
            mv = jnp.concatenate([mv, vc[n - h:h]], axis=0)
        dc, vc = md, mv
        n = h
    return dc, vc


def _tc_body(t_ref, x_ref, d_ref, v_ref, bd, bv):
    i = pl.program_id(0)
    x = x_ref[...]
    t = t_ref[...]
    dist = jnp.sum(jnp.abs(x - t[None, :, :]), axis=1)
    val = x[:, 0, :]
    dc, vc = _pairwise_min(dist, val)

    @pl.when(i == 0)
    def _init():
        bd[...] = jnp.full((1, _Q), jnp.inf, jnp.float32)
        bv[...] = jnp.zeros((1, _Q), jnp.float32)

    pd = bd[...]
    pv = bv[...]
    w = dc < pd
    bd[...] = jnp.where(w, dc, pd)
    bv[...] = jnp.where(w, vc, pv)

    @pl.when(i == _TSTEPS - 1)
    def _emit():
        d_ref[...] = bd[...]
        v_ref[...] = bv[...]


def _tc_call(agg_t, tensor_t):
    return pl.pallas_call(
        _tc_body,
        grid=(_TSTEPS,),
        in_specs=[
            pl.BlockSpec((_D, _Q), lambda i: (0, 0)),
            pl.BlockSpec((_KBT, _D, _Q), lambda i: (_K_SC // _KBT + i, 0, 0)),
        ],
        out_specs=[
            pl.BlockSpec((1, _Q), lambda i: (0, 0)),
            pl.BlockSpec((1, _Q), lambda i: (0, 0)),
        ],
        out_shape=(jax.ShapeDtypeStruct((1, _Q), jnp.float32),
                   jax.ShapeDtypeStruct((1, _Q), jnp.float32)),
        scratch_shapes=[
            pltpu.VMEM((1, _Q), jnp.float32),
            pltpu.VMEM((1, _Q), jnp.float32),
        ],
    )(tensor_t, agg_t)


def _merge_body(sd_ref, sv_ref, td_ref, tv_ref, o_ref):
    sd = sd_ref[...]
    sv = sv_ref[...]
    w = td_ref[...] < sd
    o_ref[...] = jnp.where(w, tv_ref[...], sv)


def _merge_call(sc_d, sc_v, tc_d, tc_v):
    return pl.pallas_call(
        _merge_body,
        out_shape=jax.ShapeDtypeStruct((1, _Q), jnp.float32),
    )(sc_d, sc_v, tc_d, tc_v)


@jax.jit
def _run(agg_t, tensor_t):
    sc_d, sc_v = _sc_call(agg_t, tensor_t)
    tc_d, tc_v = _tc_call(agg_t, tensor_t)
    merged = _merge_call(sc_d.reshape(1, _Q), sc_v.reshape(1, _Q), tc_d, tc_v)
    return merged


def kernel(tensor, aggregated_values):
    k, q, d = aggregated_values.shape
    assert (k, q, d) == (_K, _Q, _D)
    agg_t = jnp.transpose(aggregated_values, (0, 2, 1))
    tensor_t = tensor.T
    out = _run(agg_t, tensor_t)
    return out.reshape(1, _Q, 1)


# speedup vs baseline: 2.1790x; 1.0310x over previous
import functools

import jax
import jax.numpy as jnp
from jax import lax
from jax.experimental import pallas as pl
from jax.experimental.pallas import tpu as pltpu, tpu_sc as plsc

_NC = 2
_NS = 16
_L = 16

_K = 1000
_Q = 1024
_D = 16

_K_SC = 400
_K_TC = _K - _K_SC

_NQB = 8
_QB = _Q // _NQB
_NG = _QB // _L
_NJ = 4
_KJ = _K_SC // _NJ
_KB = 25
_NCHUNK = _KJ // _KB

_KBT = 50
_TSTEPS = _K_TC // _KBT


def _sc_body(agg_hbm, t_hbm, outd_hbm, outv_hbm, buf, tvm, stage, merged,
             outd, outv, shared, sems):
    c = lax.axis_index("c")
    s = lax.axis_index("s")
    b = c * (_NQB // _NC) + s // _NJ
    j = s % _NJ
    q0 = b * _QB
    k_base = j * _KJ

    pltpu.sync_copy(t_hbm.at[:, pl.ds(q0, _QB)], tvm)

    def src(ci):
        return agg_hbm.at[pl.ds(k_base + ci * _KB, _KB), :, pl.ds(q0, _QB)]

    inf = jnp.full((_L,), jnp.inf, jnp.float32)
    zero = jnp.zeros((_L,), jnp.float32)
    carry = (inf, zero) * _NG

    pltpu.async_copy(src(0), buf.at[0], sems.at[0])

    def chunk_body(ci, carry):
        par = lax.rem(ci, 2)
        cur = buf.at[par]
        pltpu.make_async_copy(src(ci), cur, sems.at[par]).wait()

        @pl.when(ci + 1 < _NCHUNK)
        def _prefetch():
            pltpu.async_copy(src(ci + 1), buf.at[1 - par], sems.at[1 - par])

        out = list(carry)
        for g in range(_NG):
            tg = [tvm[d, pl.ds(g * _L, _L)] for d in range(_D)]

            def k_body(k, st, g=g, tg=tg, cur=cur):
                best, bval = st
                v0 = None
                acc_a = None
                acc_b = None
                for d in range(_D):
                    v = cur[k, d, pl.ds(g * _L, _L)]
                    if d == 0:
                        v0 = v
                    term = jnp.abs(v - tg[d])
                    if d % 2 == 0:
                        acc_a = term if acc_a is None else acc_a + term
                    else:
                        acc_b = term if acc_b is None else acc_b + term
                dist = acc_a + acc_b
                better = dist < best
                return (jnp.where(better, dist, best),
                        jnp.where(better, v0, bval))

            out[2 * g], out[2 * g + 1] = lax.fori_loop(
                0, _KB, k_body, (out[2 * g], out[2 * g + 1]), unroll=5)
        return tuple(out)

    carry = lax.fori_loop(0, _NCHUNK, chunk_body, carry)

    for g in range(_NG):
        stage[0, 0, pl.ds(g * _L, _L)] = carry[2 * g]
        stage[0, 0, pl.ds(_QB + g * _L, _L)] = carry[2 * g + 1]
    pltpu.sync_copy(stage, shared.at[pl.ds(s, 1)])
    plsc.subcore_barrier()

    @pl.when(j == 0)
    def _merge():
        pltpu.sync_copy(shared.at[pl.ds(s, _NJ)], merged)
        for g in range(_NG):
            bd = merged[0, 0, pl.ds(g * _L, _L)]
            bv = merged[0, 0, pl.ds(_QB + g * _L, _L)]
            for jj in range(1, _NJ):
                dd = merged[jj, 0, pl.ds(g * _L, _L)]
                vv = merged[jj, 0, pl.ds(_QB + g * _L, _L)]
                m = dd < bd
                bd = jnp.where(m, dd, bd)
                bv = jnp.where(m, vv, bv)
            outd[0, 0, pl.ds(g * _L, _L)] = bd
            outv[0, 0, pl.ds(g * _L, _L)] = bv
        pltpu.sync_copy(outd, outd_hbm.at[pl.ds(b, 1)])
        pltpu.sync_copy(outv, outv_hbm.at[pl.ds(b, 1)])


def _sc_call(agg_t, tensor_t):
    mesh = plsc.VectorSubcoreMesh(
        core_axis_name="c", subcore_axis_name="s",
        num_cores=_NC, num_subcores=_NS)
    return pl.kernel(
        _sc_body,
        out_type=(jax.ShapeDtypeStruct((_NQB, 1, _QB), jnp.float32),
                  jax.ShapeDtypeStruct((_NQB, 1, _QB), jnp.float32)),
        mesh=mesh,
        scratch_types=[
            pltpu.VMEM((2, _KB, _D, _QB), jnp.float32),
            pltpu.VMEM((_D, _QB), jnp.float32),
            pltpu.VMEM((1, 1, 2 * _QB), jnp.float32),
            pltpu.VMEM((_NJ, 1, 2 * _QB), jnp.float32),
            pltpu.VMEM((1, 1, _QB), jnp.float32),
            pltpu.VMEM((1, 1, _QB), jnp.float32),
            pltpu.VMEM_SHARED((_NS, 1, 2 * _QB), jnp.float32),
            pltpu.SemaphoreType.DMA((2,)),
        ],
    )(agg_t, tensor_t)


def _pairwise_min(dc, vc):
    n = dc.shape[0]
    while n > 1:
        h = (n + 1) // 2
        a_d, a_v = dc[: n - h], vc[: n - h]
        b_d, b_v = dc[h:n], vc[h:n]
        w = b_d < a_d
        md = jnp.where(w, b_d, a_d)
        mv = jnp.where(w, b_v, a_v)
        if h > n - h:
            md = jnp.concatenate([md, dc[n - h:h]], axis=0)
            mv = jnp.concatenate([mv, vc[n - h:h]], axis=0)
        dc, vc = md, mv
        n = h
    return dc, vc


def _tc_body(t_ref, x_ref, d_ref, v_ref, bd, bv):
    i = pl.program_id(0)
    x = x_ref[...]
    t = t_ref[...]
    dist = jnp.sum(jnp.abs(x - t[None, :, :]), axis=1)
    val = x[:, 0, :]
    dc, vc = _pairwise_min(dist, val)

    @pl.when(i == 0)
    def _init():
        bd[...] = jnp.full((1, _Q), jnp.inf, jnp.float32)
        bv[...] = jnp.zeros((1, _Q), jnp.float32)

    pd = bd[...]
    pv = bv[...]
    w = dc < pd
    bd[...] = jnp.where(w, dc, pd)
    bv[...] = jnp.where(w, vc, pv)

    @pl.when(i == _TSTEPS - 1)
    def _emit():
        d_ref[...] = bd[...]
        v_ref[...] = bv[...]


def _tc_call(agg_t, tensor_t):
    return pl.pallas_call(
        _tc_body,
        grid=(_TSTEPS,),
        in_specs=[
            pl.BlockSpec((_D, _Q), lambda i: (0, 0)),
            pl.BlockSpec((_KBT, _D, _Q), lambda i: (_K_SC // _KBT + i, 0, 0)),
        ],
        out_specs=[
            pl.BlockSpec((1, _Q), lambda i: (0, 0)),
            pl.BlockSpec((1, _Q), lambda i: (0, 0)),
        ],
        out_shape=(jax.ShapeDtypeStruct((1, _Q), jnp.float32),
                   jax.ShapeDtypeStruct((1, _Q), jnp.float32)),
        scratch_shapes=[
            pltpu.VMEM((1, _Q), jnp.float32),
            pltpu.VMEM((1, _Q), jnp.float32),
        ],
    )(tensor_t, agg_t)


def _merge_body(sd_ref, sv_ref, td_ref, tv_ref, o_ref):
    sd = sd_ref[...]
    sv = sv_ref[...]
    w = td_ref[...] < sd
    o_ref[...] = jnp.where(w, tv_ref[...], sv)


def _merge_call(sc_d, sc_v, tc_d, tc_v):
    return pl.pallas_call(
        _merge_body,
        out_shape=jax.ShapeDtypeStruct((1, _Q), jnp.float32),
    )(sc_d, sc_v, tc_d, tc_v)


@jax.jit
def _run(agg_t, tensor_t):
    sc_d, sc_v = _sc_call(agg_t, tensor_t)
    tc_d, tc_v = _tc_call(agg_t, tensor_t)
    merged = _merge_call(sc_d.reshape(1, _Q), sc_v.reshape(1, _Q), tc_d, tc_v)
    return merged


def kernel(tensor, aggregated_values):
    k, q, d = aggregated_values.shape
    assert (k, q, d) == (_K, _Q, _D)
    agg_t = jnp.transpose(aggregated_values, (0, 2, 1))
    tensor_t = tensor.T
    out = _run(agg_t, tensor_t)
    return out.reshape(1, _Q, 1)
